# baseline (speedup 1.0000x reference)
import jax
import jax.numpy as jnp
from jax.experimental import pallas as pl
from jax.experimental.pallas import tpu as pltpu

_P_STEPS = 12
_TAIL = 6
_F32 = jnp.float32


def _prep_kernel(attr_ref, md_ref, mk_ref, wpe1_ref, bpe1_ref, wpe2_ref,
                 bpe2_ref, w_ref):
    attr = attr_ref[0]
    EA, R, N = attr.shape
    pe1 = jnp.tanh(
        jax.lax.dot_general(wpe1_ref[...], attr.reshape(EA, R * N),
                            (((0,), (0,)), ((), ())),
                            preferred_element_type=_F32)
        + bpe1_ref[...])
    pe = jax.lax.dot_general(wpe2_ref[...], pe1, (((1,), (0,)), ((), ())),
                             preferred_element_type=_F32)
    pe = pe.reshape(R, N) + bpe2_ref[0, 0]
    m = jnp.clip(md_ref[0] + mk_ref[0], 0.0, 1.0)
    w_ref[0] = jax.nn.sigmoid(pe) * m


def _dotT(a, b):
    return jax.lax.dot_general(a, b, (((0,), (0,)), ((), ())),
                               preferred_element_type=_F32)


def _mm(a, b):
    return jax.lax.dot_general(a, b, (((1,), (0,)), ((), ())),
                               preferred_element_type=_F32)


def _gates(gx, gh, h, H):
    rz = jax.nn.sigmoid(gx[:, :2 * H] + gh[:, :2 * H])
    r = rz[:, :H]
    z = rz[:, H:]
    n = jnp.tanh(gx[:, 2 * H:] + r * gh[:, 2 * H:])
    return n + z * (h - n)


def _make_recur(B, N, T, H, L):
    BN = B * N

    def body(xt_ref, nattr_ref, w_ref, outlet_ref, win_ref, bin_ref,
             wfilm_ref, bfilm_ref, enc_w_ref, dec_w_ref, wout_ref, bout_ref,
             wfb_ref, out_ref, ahat_ref, gx0_ref, pred_ref):
        for b in range(B):
            wb = w_ref[b]
            inv = 1.0 / (jnp.sum(wb, axis=0) + 1e-6)
            for bb in range(B):
                ahat_ref[bb * N:(bb + 1) * N, b * N:(b + 1) * N] = (
                    wb * inv[None, :] if bb == b else jnp.zeros((N, N), _F32))
        A = ahat_ref[...]

        xt = xt_ref[...]
        F = xt.shape[2]
        hp = _mm(xt.reshape(T * BN, F), win_ref[...]) + bin_ref[...]
        film = _mm(nattr_ref[...], wfilm_ref[...]) + bfilm_ref[...]
        hp = hp.reshape(T, BN, H)
        hp = hp * (1.0 + film[None, :, :H]) + film[None, :, H:]

        gx0_ref[...] = (_mm(hp.reshape(T * BN, H),
                            enc_w_ref[0, :H, :]) + enc_w_ref[0, 2 * H, :]
                        ).reshape(T, BN, 3 * H)

        zeros = jnp.zeros((BN, H), _F32)

        def enc_body(t, carry):
            h0, h1, acc = carry
            agg0 = _dotT(A, h0)
            gh0 = _mm(agg0, enc_w_ref[0, H:2 * H, :])
            h0 = _gates(gx0_ref[t], gh0, h0, H)
            agg1 = _dotT(A, h1)
            gx1 = _mm(h0, enc_w_ref[1, :H, :]) + enc_w_ref[1, 2 * H, :]
            gh1 = _mm(agg1, enc_w_ref[1, H:2 * H, :])
            h1 = _gates(gx1, gh1, h1, H)
            acc = acc + jnp.where(t >= T - _TAIL, 1.0, 0.0) * h1
            return h0, h1, acc

        h0, h1, acc = jax.lax.fori_loop(0, T, enc_body, (zeros, zeros, zeros))
        context = acc * (1.0 / _TAIL)

        wfb = wfb_ref[...]
        wout = wout_ref[...]
        bout = bout_ref[0, 0]

        def dec_body(p, carry):
            h0, h1, y = carry
            inp = context + y * wfb
            agg0 = _dotT(A, h0)
            gx0 = _mm(inp, dec_w_ref[0, :H, :]) + dec_w_ref[0, 2 * H, :]
            gh0 = _mm(agg0, dec_w_ref[0, H:2 * H, :])
            h0 = _gates(gx0, gh0, h0, H)
            agg1 = _dotT(A, h1)
            gx1 = _mm(h0, dec_w_ref[1, :H, :]) + dec_w_ref[1, 2 * H, :]
            gh1 = _mm(agg1, dec_w_ref[1, H:2 * H, :])
            h1 = _gates(gx1, gh1, h1, H)
            y = jnp.sum(h1 * wout, axis=1, keepdims=True) + bout
            pred_ref[p, :] = y[:, 0]
            return h0, h1, y

        jax.lax.fori_loop(0, _P_STEPS, dec_body,
                          (h0, h1, jnp.zeros((BN, 1), _F32)))

        pred = pred_ref[...]
        K = outlet_ref.shape[-1]
        for b in range(B):
            outlet = outlet_ref[b, 0]
            iota = jax.lax.broadcasted_iota(jnp.int32, (K, N), 1)
            onehot = (iota == outlet[:, None]).astype(_F32)
            out_ref[b] = jax.lax.dot_general(
                pred[:, b * N:(b + 1) * N], onehot, (((1,), (1,)), ((), ())),
                preferred_element_type=_F32)

    return body


def kernel(x, node_attr, mask_downstream_adj, mask_khop_up_adj,
           full_path_edge_attr_adj, outlet_index, params):
    B, N, T, F = x.shape
    NA = node_attr.shape[-1]
    EA = full_path_edge_attr_adj.shape[-1]
    PH = params["W_pe1"].shape[1]
    H = params["W_in"].shape[1]
    K = outlet_index.shape[-1]
    L = sum(1 for k in params if k.startswith("enc_Wx_"))
    BN = B * N

    R = 64
    attr_t = jnp.transpose(full_path_edge_attr_adj, (0, 3, 1, 2))
    w = pl.pallas_call(
        _prep_kernel,
        grid=(B, N // R),
        in_specs=[
            pl.BlockSpec((1, EA, R, N), lambda b, r: (b, 0, r, 0)),
            pl.BlockSpec((1, R, N), lambda b, r: (b, r, 0)),
            pl.BlockSpec((1, R, N), lambda b, r: (b, r, 0)),
            pl.BlockSpec((EA, PH), lambda b, r: (0, 0)),
            pl.BlockSpec((PH, 1), lambda b, r: (0, 0)),
            pl.BlockSpec((1, PH), lambda b, r: (0, 0)),
            pl.BlockSpec((1, 1), lambda b, r: (0, 0)),
        ],
        out_specs=pl.BlockSpec((1, R, N), lambda b, r: (b, r, 0)),
        out_shape=jax.ShapeDtypeStruct((B, N, N), _F32),
        compiler_params=pltpu.CompilerParams(
            dimension_semantics=("parallel", "parallel")),
    )(attr_t, mask_downstream_adj, mask_khop_up_adj,
      params["W_pe1"], params["b_pe1"].reshape(PH, 1),
      params["W_pe2"].reshape(1, PH), params["b_pe2"].reshape(1, 1))

    xt = jnp.transpose(x, (2, 0, 1, 3)).reshape(T, BN, F)
    nattr2 = node_attr.reshape(BN, NA)
    outlet3 = outlet_index.reshape(B, 1, K)

    def stack_gru(tag):
        mats = []
        for l in range(L):
            mats.append(jnp.concatenate(
                [params[f"{tag}_Wx_{l}"], params[f"{tag}_Wh_{l}"],
                 params[f"{tag}_b_{l}"].reshape(1, 3 * H)], axis=0))
        return jnp.stack(mats, axis=0)

    enc_w = stack_gru("enc")
    dec_w = stack_gru("dec")

    operands = [
        xt, nattr2, w, outlet3,
        params["W_in"], params["b_in"].reshape(1, H),
        params["W_film"], params["b_film"].reshape(1, 2 * H),
        enc_w, dec_w,
        params["W_out"].reshape(1, H), params["b_out"].reshape(1, 1),
        params["W_fb"],
    ]

    out = pl.pallas_call(
        _make_recur(B, N, T, H, L),
        out_shape=jax.ShapeDtypeStruct((B, _P_STEPS, K), _F32),
        scratch_shapes=[pltpu.VMEM((BN, BN), _F32),
                        pltpu.VMEM((T, BN, 3 * H), _F32),
                        pltpu.VMEM((_P_STEPS, BN), _F32)],
    )(*operands)
    return out


# speedup vs baseline: 1.3348x; 1.3348x over previous
import jax
import jax.numpy as jnp
from jax.experimental import pallas as pl
from jax.experimental.pallas import tpu as pltpu

_P_STEPS = 12
_TAIL = 6
_F32 = jnp.float32


def _prep_kernel(attr_ref, md_ref, mk_ref, wpe1_ref, bpe1_ref, wpe2_ref,
                 bpe2_ref, w_ref):
    attr = attr_ref[0]
    EA, R, N = attr.shape
    pe1 = jnp.tanh(
        jax.lax.dot_general(wpe1_ref[...], attr.reshape(EA, R * N),
                            (((0,), (0,)), ((), ())),
                            preferred_element_type=_F32)
        + bpe1_ref[...])
    pe = jax.lax.dot_general(wpe2_ref[...], pe1, (((1,), (0,)), ((), ())),
                             preferred_element_type=_F32)
    pe = pe.reshape(R, N) + bpe2_ref[0, 0]
    m = jnp.clip(md_ref[0] + mk_ref[0], 0.0, 1.0)
    w_ref[0] = jax.nn.sigmoid(pe) * m


def _dotT(a, b):
    return jax.lax.dot_general(a, b, (((0,), (0,)), ((), ())),
                               preferred_element_type=_F32)


def _mm(a, b):
    return jax.lax.dot_general(a, b, (((1,), (0,)), ((), ())),
                               preferred_element_type=_F32)


def _gates(gx, gh, h, H):
    rz = jax.nn.sigmoid(gx[:, :2 * H] + gh[:, :2 * H])
    r = rz[:, :H]
    z = rz[:, H:]
    n = jnp.tanh(gx[:, 2 * H:] + r * gh[:, 2 * H:])
    return n + z * (h - n)


def _make_recur(B, N, T, H):
    BN = B * N

    def body(xt_ref, nattr_ref, w_ref, outlet_ref, win_ref, bin_ref,
             wfilm_ref, bfilm_ref, enc_w_ref, dec_w_ref, wout_ref, bout_ref,
             wfb_ref, m2_ref, cfb_ref, out_ref, gx0_ref, pred_ref):
        As = []
        for b in range(B):
            wb = w_ref[b]
            inv = 1.0 / (jnp.sum(wb, axis=0) + 1e-6)
            As.append(wb * inv[None, :])

        def conv(h):
            return jnp.concatenate(
                [_dotT(As[b], h[b * N:(b + 1) * N, :]) for b in range(B)],
                axis=0)

        xt = xt_ref[...]
        F = xt.shape[2]
        hp = _mm(xt.reshape(T * BN, F), win_ref[...]) + bin_ref[...]
        film = _mm(nattr_ref[...], wfilm_ref[...]) + bfilm_ref[...]
        hp = hp.reshape(T, BN, H)
        hp = hp * (1.0 + film[None, :, :H]) + film[None, :, H:]

        gx0_ref[...] = (_mm(hp.reshape(T * BN, H),
                            enc_w_ref[0, :H, :]) + enc_w_ref[0, 2 * H, :]
                        ).reshape(T, BN, 3 * H)

        zeros = jnp.zeros((BN, H), _F32)

        def enc_body(t, carry):
            h0, h1, acc = carry
            gh0 = _mm(conv(h0), enc_w_ref[0, H:2 * H, :])
            h0 = _gates(gx0_ref[t], gh0, h0, H)
            agg1 = conv(h1)
            gx1 = _mm(h0, enc_w_ref[1, :H, :]) + enc_w_ref[1, 2 * H, :]
            gh1 = _mm(agg1, enc_w_ref[1, H:2 * H, :])
            h1 = _gates(gx1, gh1, h1, H)
            acc = acc + jnp.where(t >= T - _TAIL, 1.0, 0.0) * h1
            return h0, h1, acc

        h0, h1, acc = jax.lax.fori_loop(0, T, enc_body, (zeros, zeros, zeros))
        context = acc * (1.0 / _TAIL)

        c0 = _mm(context, dec_w_ref[0, :H, :]) + dec_w_ref[0, 2 * H, :]
        wout = wout_ref[...]
        bout = bout_ref[0, 0]

        def dec_body(p, carry):
            h0, h1, fb = carry
            gh0 = _mm(conv(h0), dec_w_ref[0, H:2 * H, :])
            h0 = _gates(c0 + fb, gh0, h0, H)
            agg1 = conv(h1)
            gx1 = _mm(h0, dec_w_ref[1, :H, :]) + dec_w_ref[1, 2 * H, :]
            gh1 = _mm(agg1, dec_w_ref[1, H:2 * H, :])
            h1 = _gates(gx1, gh1, h1, H)
            fb = _mm(h1, m2_ref[...]) + cfb_ref[...]
            y = jnp.sum(h1 * wout, axis=1, keepdims=True) + bout
            pred_ref[p, :] = y[:, 0]
            return h0, h1, fb

        jax.lax.fori_loop(0, _P_STEPS, dec_body,
                          (h0, h1, jnp.zeros((BN, 3 * H), _F32)))

        pred = pred_ref[...]
        K = outlet_ref.shape[-1]
        for b in range(B):
            outlet = outlet_ref[b, 0]
            iota = jax.lax.broadcasted_iota(jnp.int32, (K, N), 1)
            onehot = (iota == outlet[:, None]).astype(_F32)
            out_ref[b] = jax.lax.dot_general(
                pred[:, b * N:(b + 1) * N], onehot, (((1,), (1,)), ((), ())),
                preferred_element_type=_F32)

    return body


def kernel(x, node_attr, mask_downstream_adj, mask_khop_up_adj,
           full_path_edge_attr_adj, outlet_index, params):
    B, N, T, F = x.shape
    NA = node_attr.shape[-1]
    EA = full_path_edge_attr_adj.shape[-1]
    PH = params["W_pe1"].shape[1]
    H = params["W_in"].shape[1]
    K = outlet_index.shape[-1]
    L = sum(1 for k in params if k.startswith("enc_Wx_"))
    assert L == 2
    BN = B * N

    R = 64
    attr_t = jnp.transpose(full_path_edge_attr_adj, (0, 3, 1, 2))
    w = pl.pallas_call(
        _prep_kernel,
        grid=(B, N // R),
        in_specs=[
            pl.BlockSpec((1, EA, R, N), lambda b, r: (b, 0, r, 0)),
            pl.BlockSpec((1, R, N), lambda b, r: (b, r, 0)),
            pl.BlockSpec((1, R, N), lambda b, r: (b, r, 0)),
            pl.BlockSpec((EA, PH), lambda b, r: (0, 0)),
            pl.BlockSpec((PH, 1), lambda b, r: (0, 0)),
            pl.BlockSpec((1, PH), lambda b, r: (0, 0)),
            pl.BlockSpec((1, 1), lambda b, r: (0, 0)),
        ],
        out_specs=pl.BlockSpec((1, R, N), lambda b, r: (b, r, 0)),
        out_shape=jax.ShapeDtypeStruct((B, N, N), _F32),
        compiler_params=pltpu.CompilerParams(
            dimension_semantics=("parallel", "parallel")),
    )(attr_t, mask_downstream_adj, mask_khop_up_adj,
      params["W_pe1"], params["b_pe1"].reshape(PH, 1),
      params["W_pe2"].reshape(1, PH), params["b_pe2"].reshape(1, 1))

    xt = jnp.transpose(x, (2, 0, 1, 3)).reshape(T, BN, F)
    nattr2 = node_attr.reshape(BN, NA)
    outlet3 = outlet_index.reshape(B, 1, K)

    def stack_gru(tag):
        mats = []
        for l in range(L):
            mats.append(jnp.concatenate(
                [params[f"{tag}_Wx_{l}"], params[f"{tag}_Wh_{l}"],
                 params[f"{tag}_b_{l}"].reshape(1, 3 * H)], axis=0))
        return jnp.stack(mats, axis=0)

    enc_w = stack_gru("enc")
    dec_w = stack_gru("dec")
    wfb_wx0 = params["W_fb"] @ params["dec_Wx_0"]
    m2 = params["W_out"] @ wfb_wx0
    cfb = params["b_out"].reshape(1, 1) * wfb_wx0

    operands = [
        xt, nattr2, w, outlet3,
        params["W_in"], params["b_in"].reshape(1, H),
        params["W_film"], params["b_film"].reshape(1, 2 * H),
        enc_w, dec_w,
        params["W_out"].reshape(1, H), params["b_out"].reshape(1, 1),
        params["W_fb"], m2, cfb,
    ]

    out = pl.pallas_call(
        _make_recur(B, N, T, H),
        out_shape=jax.ShapeDtypeStruct((B, _P_STEPS, K), _F32),
        scratch_shapes=[pltpu.VMEM((T, BN, 3 * H), _F32),
                        pltpu.VMEM((_P_STEPS, BN), _F32)],
    )(*operands)
    return out


# speedup vs baseline: 1.4644x; 1.0972x over previous
import jax
import jax.numpy as jnp
from jax.experimental import pallas as pl
from jax.experimental.pallas import tpu as pltpu

_P_STEPS = 12
_TAIL = 6
_F32 = jnp.float32


def _prep_kernel(attr_ref, md_ref, mk_ref, wpe1_ref, bpe1_ref, wpe2_ref,
                 bpe2_ref, w_ref):
    attr = attr_ref[0]
    EA, R, N = attr.shape
    pe1 = jnp.tanh(
        jax.lax.dot_general(wpe1_ref[...], attr.reshape(EA, R * N),
                            (((0,), (0,)), ((), ())),
                            preferred_element_type=_F32)
        + bpe1_ref[...])
    pe = jax.lax.dot_general(wpe2_ref[...], pe1, (((1,), (0,)), ((), ())),
                             preferred_element_type=_F32)
    pe = pe.reshape(R, N) + bpe2_ref[0, 0]
    m = jnp.clip(md_ref[0] + mk_ref[0], 0.0, 1.0)
    w_ref[0] = jax.nn.sigmoid(pe) * m


def _dotT(a, b):
    return jax.lax.dot_general(a, b, (((0,), (0,)), ((), ())),
                               preferred_element_type=_F32)


def _mm(a, b):
    return jax.lax.dot_general(a, b, (((1,), (0,)), ((), ())),
                               preferred_element_type=_F32)


def _gates_c(c, h, H):
    rz = jax.nn.sigmoid(c[:, :2 * H])
    n = jnp.tanh(c[:, 2 * H:3 * H] + rz[:, :H] * c[:, 3 * H:])
    return n + rz[:, H:2 * H] * (h - n)


def _make_recur(B, N, T, H):
    BN = B * N

    def body(xt_ref, nattr_ref, w_ref, outlet_ref, win_ref, bin_ref,
             wfilm_ref, bfilm_ref, enc0_ref, enc1_ref, dec0h_ref, dec1_ref,
             dec0x_ref, bout_ref, fby_ref, cfb_ref, out_ref, hp_ref):
        As = []
        for b in range(B):
            wb = w_ref[b]
            inv = 1.0 / (jnp.sum(wb, axis=0) + 1e-6)
            As.append(wb * inv[None, :])

        def conv2(hcat):
            return jnp.concatenate(
                [_dotT(As[b], hcat[b * N:(b + 1) * N, :]) for b in range(B)],
                axis=0)

        xt = xt_ref[...]
        F = xt.shape[2]
        hp = _mm(xt.reshape(T * BN, F), win_ref[...]) + bin_ref[...]
        film = _mm(nattr_ref[...], wfilm_ref[...]) + bfilm_ref[...]
        hp = hp.reshape(T, BN, H)
        hp_ref[...] = hp * (1.0 + film[None, :, :H]) + film[None, :, H:]

        zeros2 = jnp.zeros((BN, 2 * H), _F32)
        zerosH = jnp.zeros((BN, H), _F32)

        def enc_body(t, carry):
            hcat, acc = carry
            aggcat = conv2(hcat)
            in0 = jnp.concatenate([hp_ref[t], aggcat[:, :H]], axis=1)
            h0 = _gates_c(_mm(in0, enc0_ref[...][:2 * H, :])
                          + enc0_ref[...][2 * H:, :].reshape(1, 4 * H),
                          hcat[:, :H], H)
            in1 = jnp.concatenate([h0, aggcat[:, H:]], axis=1)
            h1 = _gates_c(_mm(in1, enc1_ref[...][:2 * H, :])
                          + enc1_ref[...][2 * H:, :].reshape(1, 4 * H),
                          hcat[:, H:], H)
            acc = acc + jnp.where(t >= T - _TAIL, 1.0, 0.0) * h1
            return jnp.concatenate([h0, h1], axis=1), acc

        hcat, acc = jax.lax.fori_loop(0, T, enc_body, (zeros2, zerosH))
        context = acc * (1.0 / _TAIL)

        c0 = _mm(context, dec0x_ref[...][:H, :]) + dec0x_ref[...][H:, :].reshape(1, 4 * H)
        bout = bout_ref[0, 0]
        lane_iota = jax.lax.broadcasted_iota(jnp.int32, (1, 128), 1)

        def dec_body(p, carry):
            hcat, fb, predv = carry
            aggcat = conv2(hcat)
            gh0 = _mm(aggcat[:, :H], dec0h_ref[...][:H, :])
            h0 = _gates_c(c0 + fb + gh0, hcat[:, :H], H)
            in1 = jnp.concatenate([h0, aggcat[:, H:]], axis=1)
            h1 = _gates_c(_mm(in1, dec1_ref[...][:2 * H, :])
                          + dec1_ref[...][2 * H:, :].reshape(1, 4 * H),
                          hcat[:, H:], H)
            fby = _mm(h1, fby_ref[...])
            fb = fby[:, :4 * H] + cfb_ref[...]
            y128 = fby[:, 4 * H:]
            predv = predv + y128 * (lane_iota == p).astype(_F32)
            return jnp.concatenate([h0, h1], axis=1), fb, predv

        _, _, predv = jax.lax.fori_loop(
            0, _P_STEPS, dec_body,
            (hcat, jnp.zeros((BN, 4 * H), _F32), jnp.zeros((BN, 128), _F32)))

        K = outlet_ref.shape[-1]
        for b in range(B):
            outlet = outlet_ref[b, 0]
            iota = jax.lax.broadcasted_iota(jnp.int32, (N, K), 0)
            onehot_t = (iota == outlet[None, :]).astype(_F32)
            gat = _dotT(predv[b * N:(b + 1) * N, :], onehot_t)
            out_ref[b] = gat[:_P_STEPS, :] + bout

    return body


def _combined_gru_w(params, tag, l, H):
    wx = params[f"{tag}_Wx_{l}"]
    wh = params[f"{tag}_Wh_{l}"]
    b = params[f"{tag}_b_{l}"].reshape(1, 3 * H)
    z = jnp.zeros((H, H), _F32)
    top = jnp.concatenate([wx[:, :2 * H], wx[:, 2 * H:], z], axis=1)
    bot = jnp.concatenate([wh[:, :2 * H], z, wh[:, 2 * H:]], axis=1)
    bias = jnp.concatenate([b[:, :2 * H], b[:, 2 * H:],
                            jnp.zeros((1, H), _F32)], axis=1)
    return jnp.concatenate([top, bot, bias.reshape(1, 4 * H) *
                            jnp.ones((1, 1), _F32)], axis=0)


def kernel(x, node_attr, mask_downstream_adj, mask_khop_up_adj,
           full_path_edge_attr_adj, outlet_index, params):
    B, N, T, F = x.shape
    NA = node_attr.shape[-1]
    EA = full_path_edge_attr_adj.shape[-1]
    PH = params["W_pe1"].shape[1]
    H = params["W_in"].shape[1]
    K = outlet_index.shape[-1]
    L = sum(1 for k in params if k.startswith("enc_Wx_"))
    assert L == 2
    BN = B * N

    R = 64
    attr_t = jnp.transpose(full_path_edge_attr_adj, (0, 3, 1, 2))
    w = pl.pallas_call(
        _prep_kernel,
        grid=(B, N // R),
        in_specs=[
            pl.BlockSpec((1, EA, R, N), lambda b, r: (b, 0, r, 0)),
            pl.BlockSpec((1, R, N), lambda b, r: (b, r, 0)),
            pl.BlockSpec((1, R, N), lambda b, r: (b, r, 0)),
            pl.BlockSpec((EA, PH), lambda b, r: (0, 0)),
            pl.BlockSpec((PH, 1), lambda b, r: (0, 0)),
            pl.BlockSpec((1, PH), lambda b, r: (0, 0)),
            pl.BlockSpec((1, 1), lambda b, r: (0, 0)),
        ],
        out_specs=pl.BlockSpec((1, R, N), lambda b, r: (b, r, 0)),
        out_shape=jax.ShapeDtypeStruct((B, N, N), _F32),
        compiler_params=pltpu.CompilerParams(
            dimension_semantics=("parallel", "parallel")),
    )(attr_t, mask_downstream_adj, mask_khop_up_adj,
      params["W_pe1"], params["b_pe1"].reshape(PH, 1),
      params["W_pe2"].reshape(1, PH), params["b_pe2"].reshape(1, 1))

    xt = jnp.transpose(x, (2, 0, 1, 3)).reshape(T, BN, F)
    nattr2 = node_attr.reshape(BN, NA)
    outlet3 = outlet_index.reshape(B, 1, K)

    enc0 = _combined_gru_w(params, "enc", 0, H)
    enc1 = _combined_gru_w(params, "enc", 1, H)
    dec1 = _combined_gru_w(params, "dec", 1, H)
    wh0 = params["dec_Wh_0"]
    zH = jnp.zeros((H, H), _F32)
    dec0h = jnp.concatenate([wh0[:, :2 * H], zH, wh0[:, 2 * H:]], axis=1)
    wx0 = params["dec_Wx_0"]
    dec0x_w = jnp.concatenate([wx0[:, :2 * H], wx0[:, 2 * H:], zH], axis=1)
    b0 = params["dec_b_0"].reshape(1, 3 * H)
    dec0x_b = jnp.concatenate([b0[:, :2 * H], b0[:, 2 * H:],
                               jnp.zeros((1, H), _F32)], axis=1)
    dec0x = jnp.concatenate([dec0x_w, dec0x_b], axis=0)
    wfb_wx0 = params["W_fb"] @ wx0
    m2 = params["W_out"] @ wfb_wx0
    m2c = jnp.concatenate([m2[:, :2 * H], m2[:, 2 * H:], zH], axis=1)
    cfb_flat = params["b_out"].reshape(1, 1) * wfb_wx0
    cfb = jnp.concatenate([cfb_flat[:, :2 * H], cfb_flat[:, 2 * H:],
                           jnp.zeros((1, H), _F32)], axis=1)
    wout_rep = jnp.tile(params["W_out"], (1, 128))
    fby = jnp.concatenate([m2c, wout_rep], axis=1)

    operands = [
        xt, nattr2, w, outlet3,
        params["W_in"], params["b_in"].reshape(1, H),
        params["W_film"], params["b_film"].reshape(1, 2 * H),
        enc0, enc1, dec0h, dec1, dec0x,
        params["b_out"].reshape(1, 1), fby, cfb,
    ]

    out = pl.pallas_call(
        _make_recur(B, N, T, H),
        out_shape=jax.ShapeDtypeStruct((B, _P_STEPS, K), _F32),
        scratch_shapes=[pltpu.VMEM((T, BN, H), _F32)],
    )(*operands)
    return out


# speedup vs baseline: 1.4791x; 1.0100x over previous
import jax
import jax.numpy as jnp
from jax.experimental import pallas as pl
from jax.experimental.pallas import tpu as pltpu

_P_STEPS = 12
_TAIL = 6
_F32 = jnp.float32


def _prep_kernel(attr_ref, md_ref, mk_ref, wpe1_ref, bpe1_ref, wpe2_ref,
                 bpe2_ref, w_ref):
    attr = attr_ref[0]
    EA, R, N = attr.shape
    pe1 = jnp.tanh(
        jax.lax.dot_general(wpe1_ref[...], attr.reshape(EA, R * N),
                            (((0,), (0,)), ((), ())),
                            preferred_element_type=_F32)
        + bpe1_ref[...])
    pe = jax.lax.dot_general(wpe2_ref[...], pe1, (((1,), (0,)), ((), ())),
                             preferred_element_type=_F32)
    pe = pe.reshape(R, N) + bpe2_ref[0, 0]
    m = jnp.clip(md_ref[0] + mk_ref[0], 0.0, 1.0)
    w_ref[0] = jax.nn.sigmoid(pe) * m


def _dotT(a, b):
    return jax.lax.dot_general(a, b, (((0,), (0,)), ((), ())),
                               preferred_element_type=_F32)


def _mm(a, b):
    return jax.lax.dot_general(a, b, (((1,), (0,)), ((), ())),
                               preferred_element_type=_F32)


def _gates_c(c, h, H):
    rz = jax.nn.sigmoid(c[:, :2 * H])
    n = jnp.tanh(c[:, 2 * H:3 * H] + rz[:, :H] * c[:, 3 * H:])
    return n + rz[:, H:2 * H] * (h - n)


def _make_recur(B, N, T, H):
    BN = B * N

    def body(xt_ref, nattr_ref, w_ref, outlet_ref, win_ref, bin_ref,
             wfilm_ref, bfilm_ref, enc0_ref, enc1_ref, dec0h_ref, dec1_ref,
             dec0x_ref, bout_ref, fby_ref, cfb_ref, out_ref, hp_ref):
        As = []
        for b in range(B):
            wtb = w_ref[b]
            inv = 1.0 / (jnp.sum(wtb, axis=1) + 1e-6)
            As.append(wtb * inv[:, None])

        def conv2(hcat):
            return jnp.concatenate(
                [_mm(As[b], hcat[b * N:(b + 1) * N, :]) for b in range(B)],
                axis=0)

        xt = xt_ref[...]
        F = xt.shape[2]
        hp = _mm(xt.reshape(T * BN, F), win_ref[...]) + bin_ref[...]
        film = _mm(nattr_ref[...], wfilm_ref[...]) + bfilm_ref[...]
        hp = hp.reshape(T, BN, H)
        hp_ref[...] = hp * (1.0 + film[None, :, :H]) + film[None, :, H:]

        zeros2 = jnp.zeros((BN, 2 * H), _F32)
        zerosH = jnp.zeros((BN, H), _F32)

        def enc_body(t, carry):
            hcat, acc = carry
            aggcat = conv2(hcat)
            in0 = jnp.concatenate([hp_ref[t], aggcat[:, :H]], axis=1)
            h0 = _gates_c(_mm(in0, enc0_ref[...][:2 * H, :])
                          + enc0_ref[...][2 * H:, :].reshape(1, 4 * H),
                          hcat[:, :H], H)
            in1 = jnp.concatenate([h0, aggcat[:, H:]], axis=1)
            h1 = _gates_c(_mm(in1, enc1_ref[...][:2 * H, :])
                          + enc1_ref[...][2 * H:, :].reshape(1, 4 * H),
                          hcat[:, H:], H)
            acc = acc + jnp.where(t >= T - _TAIL, 1.0, 0.0) * h1
            return jnp.concatenate([h0, h1], axis=1), acc

        hcat, acc = jax.lax.fori_loop(0, T, enc_body, (zeros2, zerosH))
        context = acc * (1.0 / _TAIL)

        c0 = _mm(context, dec0x_ref[...][:H, :]) + dec0x_ref[...][H:, :].reshape(1, 4 * H)
        bout = bout_ref[0, 0]
        lane_iota = jax.lax.broadcasted_iota(jnp.int32, (1, 128), 1)

        def dec_body(p, carry):
            hcat, fb, predv = carry
            aggcat = conv2(hcat)
            gh0 = _mm(aggcat[:, :H], dec0h_ref[...][:H, :])
            h0 = _gates_c(c0 + fb + gh0, hcat[:, :H], H)
            in1 = jnp.concatenate([h0, aggcat[:, H:]], axis=1)
            h1 = _gates_c(_mm(in1, dec1_ref[...][:2 * H, :])
                          + dec1_ref[...][2 * H:, :].reshape(1, 4 * H),
                          hcat[:, H:], H)
            fby = _mm(h1, fby_ref[...])
            fb = fby[:, :4 * H] + cfb_ref[...]
            y128 = fby[:, 4 * H:]
            predv = predv + y128 * (lane_iota == p).astype(_F32)
            return jnp.concatenate([h0, h1], axis=1), fb, predv

        _, _, predv = jax.lax.fori_loop(
            0, _P_STEPS, dec_body,
            (hcat, jnp.zeros((BN, 4 * H), _F32), jnp.zeros((BN, 128), _F32)))

        K = outlet_ref.shape[-1]
        for b in range(B):
            outlet = outlet_ref[b, 0]
            iota = jax.lax.broadcasted_iota(jnp.int32, (N, K), 0)
            onehot_t = (iota == outlet[None, :]).astype(_F32)
            gat = _dotT(predv[b * N:(b + 1) * N, :], onehot_t)
            out_ref[b] = gat[:_P_STEPS, :] + bout

    return body


def _combined_gru_w(params, tag, l, H):
    wx = params[f"{tag}_Wx_{l}"]
    wh = params[f"{tag}_Wh_{l}"]
    b = params[f"{tag}_b_{l}"].reshape(1, 3 * H)
    z = jnp.zeros((H, H), _F32)
    top = jnp.concatenate([wx[:, :2 * H], wx[:, 2 * H:], z], axis=1)
    bot = jnp.concatenate([wh[:, :2 * H], z, wh[:, 2 * H:]], axis=1)
    bias = jnp.concatenate([b[:, :2 * H], b[:, 2 * H:],
                            jnp.zeros((1, H), _F32)], axis=1)
    return jnp.concatenate([top, bot, bias.reshape(1, 4 * H) *
                            jnp.ones((1, 1), _F32)], axis=0)


def kernel(x, node_attr, mask_downstream_adj, mask_khop_up_adj,
           full_path_edge_attr_adj, outlet_index, params):
    B, N, T, F = x.shape
    NA = node_attr.shape[-1]
    EA = full_path_edge_attr_adj.shape[-1]
    PH = params["W_pe1"].shape[1]
    H = params["W_in"].shape[1]
    K = outlet_index.shape[-1]
    L = sum(1 for k in params if k.startswith("enc_Wx_"))
    assert L == 2
    BN = B * N

    R = 64
    attr_t = jnp.transpose(full_path_edge_attr_adj, (0, 3, 2, 1))
    mask_d_t = jnp.transpose(mask_downstream_adj, (0, 2, 1))
    mask_k_t = jnp.transpose(mask_khop_up_adj, (0, 2, 1))
    w = pl.pallas_call(
        _prep_kernel,
        grid=(B, N // R),
        in_specs=[
            pl.BlockSpec((1, EA, R, N), lambda b, r: (b, 0, r, 0)),
            pl.BlockSpec((1, R, N), lambda b, r: (b, r, 0)),
            pl.BlockSpec((1, R, N), lambda b, r: (b, r, 0)),
            pl.BlockSpec((EA, PH), lambda b, r: (0, 0)),
            pl.BlockSpec((PH, 1), lambda b, r: (0, 0)),
            pl.BlockSpec((1, PH), lambda b, r: (0, 0)),
            pl.BlockSpec((1, 1), lambda b, r: (0, 0)),
        ],
        out_specs=pl.BlockSpec((1, R, N), lambda b, r: (b, r, 0)),
        out_shape=jax.ShapeDtypeStruct((B, N, N), _F32),
        compiler_params=pltpu.CompilerParams(
            dimension_semantics=("parallel", "parallel")),
    )(attr_t, mask_d_t, mask_k_t,
      params["W_pe1"], params["b_pe1"].reshape(PH, 1),
      params["W_pe2"].reshape(1, PH), params["b_pe2"].reshape(1, 1))

    xt = jnp.transpose(x, (2, 0, 1, 3)).reshape(T, BN, F)
    nattr2 = node_attr.reshape(BN, NA)
    outlet3 = outlet_index.reshape(B, 1, K)

    enc0 = _combined_gru_w(params, "enc", 0, H)
    enc1 = _combined_gru_w(params, "enc", 1, H)
    dec1 = _combined_gru_w(params, "dec", 1, H)
    wh0 = params["dec_Wh_0"]
    zH = jnp.zeros((H, H), _F32)
    dec0h = jnp.concatenate([wh0[:, :2 * H], zH, wh0[:, 2 * H:]], axis=1)
    wx0 = params["dec_Wx_0"]
    dec0x_w = jnp.concatenate([wx0[:, :2 * H], wx0[:, 2 * H:], zH], axis=1)
    b0 = params["dec_b_0"].reshape(1, 3 * H)
    dec0x_b = jnp.concatenate([b0[:, :2 * H], b0[:, 2 * H:],
                               jnp.zeros((1, H), _F32)], axis=1)
    dec0x = jnp.concatenate([dec0x_w, dec0x_b], axis=0)
    wfb_wx0 = params["W_fb"] @ wx0
    m2 = params["W_out"] @ wfb_wx0
    m2c = jnp.concatenate([m2[:, :2 * H], m2[:, 2 * H:], zH], axis=1)
    cfb_flat = params["b_out"].reshape(1, 1) * wfb_wx0
    cfb = jnp.concatenate([cfb_flat[:, :2 * H], cfb_flat[:, 2 * H:],
                           jnp.zeros((1, H), _F32)], axis=1)
    wout_rep = jnp.tile(params["W_out"], (1, 128))
    fby = jnp.concatenate([m2c, wout_rep], axis=1)

    operands = [
        xt, nattr2, w, outlet3,
        params["W_in"], params["b_in"].reshape(1, H),
        params["W_film"], params["b_film"].reshape(1, 2 * H),
        enc0, enc1, dec0h, dec1, dec0x,
        params["b_out"].reshape(1, 1), fby, cfb,
    ]

    out = pl.pallas_call(
        _make_recur(B, N, T, H),
        out_shape=jax.ShapeDtypeStruct((B, _P_STEPS, K), _F32),
        scratch_shapes=[pltpu.VMEM((T, BN, H), _F32)],
    )(*operands)
    return out
